# baseline (speedup 1.0000x reference)
import functools

import jax
import jax.numpy as jnp
from jax import lax
from jax.experimental import pallas as pl
from jax.experimental.pallas import tpu as pltpu
from jax.experimental.pallas import tpu_sc as plsc

L = 16
NC = 2
NS = 16
NW = NC * NS
GOUT = 4
NBUF = 4


def _rsqrt(x):
    i = lax.bitcast_convert_type(x, jnp.int32)
    y = lax.bitcast_convert_type(jnp.int32(0x5F3759DF) - (i >> 1), jnp.float32)
    for _ in range(3):
        y = y * (1.5 - 0.5 * x * y * y)
    return y


def _trig_tc_body(lon_ref, cos_ref, sin_ref):
    lon = lon_ref[...]
    cos_ref[...] = jnp.cos(lon)
    sin_ref[...] = jnp.sin(lon)


def _make_sc_kernel(B, N, D, NH, NPAD):
    CH = NPAD // NW
    WB = D // 2
    PS = (N // NS) // 8 * 8
    REM = N - PS * NS
    GPB = 2 * GOUT
    NI = CH // GPB
    mesh = plsc.VectorSubcoreMesh(core_axis_name="c", subcore_axis_name="s")

    @functools.partial(
        pl.kernel,
        mesh=mesh,
        compiler_params=pltpu.CompilerParams(needs_layout_passes=False),
        out_type=jax.ShapeDtypeStruct((B * NPAD, D), jnp.float32),
        scratch_types=[
            pltpu.VMEM_SHARED((N, WB), jnp.int32),
            pltpu.VMEM((CH * NH,), jnp.int32),
            pltpu.VMEM((CH * NH,), jnp.float32),
            [pltpu.VMEM((2 * NH, WB), jnp.int32) for _ in range(NBUF)],
            [pltpu.VMEM((GOUT, D), jnp.float32) for _ in range(2)],
            [pltpu.SemaphoreType.DMA for _ in range(NBUF)],
            [pltpu.SemaphoreType.DMA for _ in range(2)],
        ],
    )
    def sc_kernel(x_hbm, nh_hbm, lon_no_hbm, lat_no_hbm, londe_hbm,
                  latde_hbm, cosde_hbm, sinde_hbm, out_hbm,
                  spx, nh_v, w_v, xbs, obs, gsems, osems):
        sid = lax.axis_index("s")
        wid = sid * NC + lax.axis_index("c")
        n0 = wid * CH

        pltpu.sync_copy(nh_hbm.at[pl.ds(n0 * NH, CH * NH)], nh_v)

        def phase1(lon_no_v, lat_no_v, londe_v, latde_v, cosde_v, sinde_v):
            pltpu.sync_copy(lon_no_hbm, lon_no_v)
            pltpu.sync_copy(lat_no_hbm, lat_no_v)
            pltpu.sync_copy(londe_hbm.at[pl.ds(n0, CH)], londe_v)
            pltpu.sync_copy(latde_hbm.at[pl.ds(n0, CH)], latde_v)
            pltpu.sync_copy(cosde_hbm.at[pl.ds(n0, CH)], cosde_v)
            pltpu.sync_copy(sinde_hbm.at[pl.ds(n0, CH)], sinde_v)

            def wbody(j, carry):
                nh16 = nh_v[pl.ds(j * NH, NH)]
                jsp = jnp.full((L,), j, jnp.int32)
                lon_i = plsc.load_gather(lon_no_v, [nh16])
                lat_i = plsc.load_gather(lat_no_v, [nh16])
                lon_o = plsc.load_gather(londe_v, [jsp])
                lat_o = plsc.load_gather(latde_v, [jsp])
                ca = plsc.load_gather(cosde_v, [jsp])
                sa = plsc.load_gather(sinde_v, [jsp])
                dx = lon_i - lon_o
                dy = lat_i - lat_o
                rho2 = dx * dx + dy * dy
                rr = rho2 + 1e-12
                r = rr * _rsqrt(rr)
                inv_rho = _rsqrt(jnp.maximum(rho2, 1e-30))
                cosphi = (dx * ca + dy * sa) * inv_rho
                cosphi = jnp.where(rho2 > 0.0, cosphi, ca)
                lg = 0.1 * cosphi - r
                e = jnp.exp(lg - jnp.max(lg))
                w_v[pl.ds(j * NH, NH)] = e / jnp.sum(e)
                return carry

            lax.fori_loop(0, CH, wbody, 0)

        pl.run_scoped(
            phase1,
            pltpu.VMEM((N,), jnp.float32),
            pltpu.VMEM((N,), jnp.float32),
            pltpu.VMEM((CH,), jnp.float32),
            pltpu.VMEM((CH,), jnp.float32),
            pltpu.VMEM((CH,), jnp.float32),
            pltpu.VMEM((CH,), jnp.float32),
        )

        idx_even = lax.iota(jnp.int32, L) * 2
        idx_odd = idx_even + 1

        CHP = CH // 2

        def gstart(pair, xb, sem):
            pltpu.make_async_copy(
                spx.at[nh_v.at[pl.ds(pair * 2 * NH, 2 * NH)]], xb,
                sem).start()

        def gwait(xb, sem):
            pltpu.make_async_copy(
                spx.at[nh_v.at[pl.ds(0, 2 * NH)]], xb, sem).wait()

        def compute(jj, loc8, rbase, xb, ob):
            wk = [plsc.load_gather(
                      w_v, [jnp.full((L,), jj * NH + k, jnp.int32)])
                  for k in range(NH)]
            l8 = jnp.full((L,), loc8, jnp.int32)

            def cbody(c, carry):
                doff = c * 2 * L
                xe, xo = plsc.unpack(
                    plsc.bitcast(xb[rbase, pl.ds(c * L, L)], jnp.bfloat16),
                    format=plsc.PackFormat.INTERLEAVED)
                acc_e = wk[0] * xe
                acc_o = wk[0] * xo
                for k in range(1, NH):
                    xe, xo = plsc.unpack(
                        plsc.bitcast(xb[rbase + k, pl.ds(c * L, L)],
                                     jnp.bfloat16),
                        format=plsc.PackFormat.INTERLEAVED)
                    acc_e = acc_e + wk[k] * xe
                    acc_o = acc_o + wk[k] * xo
                plsc.store_scatter(ob, [l8, doff + idx_even], acc_e)
                plsc.store_scatter(ob, [l8, doff + idx_odd], acc_o)
                return carry

            lax.fori_loop(0, D // (2 * L), cbody, 0)

        def tbody(t, carry):
            b = t // NI
            i = t - b * NI

            @pl.when(i == 0)
            def _():
                plsc.subcore_barrier()
                pltpu.sync_copy(x_hbm.at[pl.ds(b * N + sid * PS, PS)],
                                spx.at[pl.ds(sid * PS, PS)])
                if REM:
                    @pl.when(sid == NS - 1)
                    def _():
                        pltpu.sync_copy(
                            x_hbm.at[pl.ds(b * N + NS * PS, REM)],
                            spx.at[pl.ds(NS * PS, REM)])
                plsc.subcore_barrier()
                for p in range(NBUF - 1):
                    gstart(p, xbs[p], gsems[p])

            for gi in range(2):
                g8 = 2 * i + gi

                @pl.when(2 * t + gi >= 2)
                def _():
                    pltpu.make_async_copy(
                        obs[gi], out_hbm.at[pl.ds(0, GOUT)],
                        osems[gi]).wait()

                for lp in range(GOUT // 2):
                    pair = i * GPB // 2 + gi * (GOUT // 2) + lp
                    par = (gi * (GOUT // 2) + lp) % NBUF

                    @pl.when(pair + NBUF - 1 < CHP)
                    def _():
                        gstart(pair + NBUF - 1, xbs[(par + NBUF - 1) % NBUF],
                               gsems[(par + NBUF - 1) % NBUF])

                    gwait(xbs[par], gsems[par])
                    for sub in range(2):
                        compute(pair * 2 + sub, lp * 2 + sub, sub * NH,
                                xbs[par], obs[gi])
                pltpu.make_async_copy(
                    obs[gi],
                    out_hbm.at[pl.ds(b * NPAD + n0 + g8 * GOUT, GOUT)],
                    osems[gi]).start()
            return carry

        lax.fori_loop(0, B * NI, tbody, 0)
        for gi in range(2):
            pltpu.make_async_copy(
                obs[gi], out_hbm.at[pl.ds(0, GOUT)], osems[gi]).wait()

    return sc_kernel


def kernel(x, coords_no, coords_decode, nh_indices):
    B, N, D = x.shape
    NH = nh_indices.shape[1]
    CH = -(-N // (NW * 2 * GOUT)) * 2 * GOUT
    NPAD = NW * CH

    nh = nh_indices.astype(jnp.int32)
    nh_p = jnp.zeros((NPAD, NH), jnp.int32).at[:N].set(nh).reshape(NPAD * NH)
    lon_no = coords_no[:, 0]
    lat_no = coords_no[:, 1]
    londe = jnp.zeros((NPAD,), jnp.float32).at[:N].set(coords_decode[:, 0])
    latde = jnp.zeros((NPAD,), jnp.float32).at[:N].set(coords_decode[:, 1])

    cosde, sinde = pl.pallas_call(
        _trig_tc_body,
        out_shape=(
            jax.ShapeDtypeStruct((NPAD // 128, 128), jnp.float32),
            jax.ShapeDtypeStruct((NPAD // 128, 128), jnp.float32),
        ),
    )(londe.reshape(NPAD // 128, 128))

    x32 = lax.bitcast_convert_type(
        x.astype(jnp.bfloat16).reshape(B * N, D // 2, 2), jnp.int32)

    sc = _make_sc_kernel(B, N, D, NH, NPAD)
    out = sc(
        x32,
        nh_p,
        lon_no,
        lat_no,
        londe,
        latde,
        cosde.reshape(NPAD),
        sinde.reshape(NPAD),
    )
    return out.reshape(B, NPAD, D)[:, :N]


# speedup vs baseline: 2.1145x; 2.1145x over previous
import functools

import jax
import jax.numpy as jnp
from jax import lax
from jax.experimental import pallas as pl
from jax.experimental.pallas import tpu as pltpu
from jax.experimental.pallas import tpu_sc as plsc

L = 16
NC = 2
NS = 16
NW = NC * NS
GOUT = 4
NBUF = 4


def _rsqrt(x):
    i = lax.bitcast_convert_type(x, jnp.int32)
    y = lax.bitcast_convert_type(jnp.int32(0x5F3759DF) - (i >> 1), jnp.float32)
    for _ in range(3):
        y = y * (1.5 - 0.5 * x * y * y)
    return y


def _trig_tc_body(lon_ref, cos_ref, sin_ref):
    lon = lon_ref[...]
    cos_ref[...] = jnp.cos(lon)
    sin_ref[...] = jnp.sin(lon)


def _pack_tc_body(x_ref, out_ref):
    half = out_ref.shape[1]
    ua = lax.bitcast_convert_type(x_ref[:, :half], jnp.uint32)
    ub = lax.bitcast_convert_type(x_ref[:, half:], jnp.uint32)
    ra = (ua + 0x7FFF + ((ua >> 16) & 1)) >> 16
    rb = (ub + 0x7FFF + ((ub >> 16) & 1)) >> 16
    out_ref[...] = lax.bitcast_convert_type(ra | (rb << 16), jnp.int32)


def _make_sc_kernel(B, N, D, NH, NPAD):
    CH = NPAD // NW
    WB = D // 2
    PS = (N // NS) // 8 * 8
    REM = N - PS * NS
    GPB = 2 * GOUT
    NI = CH // GPB
    mesh = plsc.VectorSubcoreMesh(core_axis_name="c", subcore_axis_name="s")

    @functools.partial(
        pl.kernel,
        mesh=mesh,
        compiler_params=pltpu.CompilerParams(needs_layout_passes=False),
        out_type=jax.ShapeDtypeStruct((B * NPAD, D), jnp.float32),
        scratch_types=[
            pltpu.VMEM_SHARED((N, WB), jnp.int32),
            pltpu.VMEM((CH * NH,), jnp.int32),
            pltpu.VMEM((CH * NH,), jnp.float32),
            [pltpu.VMEM((2 * NH, WB), jnp.int32) for _ in range(NBUF)],
            [pltpu.VMEM((GOUT, D), jnp.float32) for _ in range(2)],
            [pltpu.SemaphoreType.DMA for _ in range(NBUF)],
            [pltpu.SemaphoreType.DMA for _ in range(2)],
        ],
    )
    def sc_kernel(x_hbm, nh_hbm, lon_no_hbm, lat_no_hbm, londe_hbm,
                  latde_hbm, cosde_hbm, sinde_hbm, out_hbm,
                  spx, nh_v, w_v, xbs, obs, gsems, osems):
        sid = lax.axis_index("s")
        wid = sid * NC + lax.axis_index("c")
        n0 = wid * CH

        pltpu.sync_copy(nh_hbm.at[pl.ds(n0 * NH, CH * NH)], nh_v)

        def phase1(lon_no_v, lat_no_v, londe_v, latde_v, cosde_v, sinde_v):
            pltpu.sync_copy(lon_no_hbm, lon_no_v)
            pltpu.sync_copy(lat_no_hbm, lat_no_v)
            pltpu.sync_copy(londe_hbm.at[pl.ds(n0, CH)], londe_v)
            pltpu.sync_copy(latde_hbm.at[pl.ds(n0, CH)], latde_v)
            pltpu.sync_copy(cosde_hbm.at[pl.ds(n0, CH)], cosde_v)
            pltpu.sync_copy(sinde_hbm.at[pl.ds(n0, CH)], sinde_v)

            def wbody(j, carry):
                nh16 = nh_v[pl.ds(j * NH, NH)]
                jsp = jnp.full((L,), j, jnp.int32)
                lon_i = plsc.load_gather(lon_no_v, [nh16])
                lat_i = plsc.load_gather(lat_no_v, [nh16])
                lon_o = plsc.load_gather(londe_v, [jsp])
                lat_o = plsc.load_gather(latde_v, [jsp])
                ca = plsc.load_gather(cosde_v, [jsp])
                sa = plsc.load_gather(sinde_v, [jsp])
                dx = lon_i - lon_o
                dy = lat_i - lat_o
                rho2 = dx * dx + dy * dy
                rr = rho2 + 1e-12
                r = rr * _rsqrt(rr)
                inv_rho = _rsqrt(jnp.maximum(rho2, 1e-30))
                cosphi = (dx * ca + dy * sa) * inv_rho
                cosphi = jnp.where(rho2 > 0.0, cosphi, ca)
                lg = 0.1 * cosphi - r
                e = jnp.exp(lg - jnp.max(lg))
                w_v[pl.ds(j * NH, NH)] = e / jnp.sum(e)
                return carry

            lax.fori_loop(0, CH, wbody, 0)

        pl.run_scoped(
            phase1,
            pltpu.VMEM((N,), jnp.float32),
            pltpu.VMEM((N,), jnp.float32),
            pltpu.VMEM((CH,), jnp.float32),
            pltpu.VMEM((CH,), jnp.float32),
            pltpu.VMEM((CH,), jnp.float32),
            pltpu.VMEM((CH,), jnp.float32),
        )

        CHP = CH // 2

        def gstart(pair, xb, sem):
            pltpu.make_async_copy(
                spx.at[nh_v.at[pl.ds(pair * 2 * NH, 2 * NH)]], xb,
                sem).start()

        def gwait(xb, sem):
            pltpu.make_async_copy(
                spx.at[nh_v.at[pl.ds(0, 2 * NH)]], xb, sem).wait()

        def compute(jj, loc8, rbase, xb, ob):
            wk = [plsc.load_gather(
                      w_v, [jnp.full((L,), jj * NH + k, jnp.int32)])
                  for k in range(NH)]

            def cbody(c, carry):
                xa, xc = plsc.unpack(
                    plsc.bitcast(xb[rbase, pl.ds(c * L, L)], jnp.bfloat16),
                    format=plsc.PackFormat.INTERLEAVED)
                acc_a = wk[0] * xa
                acc_c = wk[0] * xc
                for k in range(1, NH):
                    xa, xc = plsc.unpack(
                        plsc.bitcast(xb[rbase + k, pl.ds(c * L, L)],
                                     jnp.bfloat16),
                        format=plsc.PackFormat.INTERLEAVED)
                    acc_a = acc_a + wk[k] * xa
                    acc_c = acc_c + wk[k] * xc
                ob[loc8, pl.ds(c * L, L)] = acc_a
                ob[loc8, pl.ds(D // 2 + c * L, L)] = acc_c
                return carry

            lax.fori_loop(0, D // (2 * L), cbody, 0)

        def tbody(t, carry):
            b = t // NI
            i = t - b * NI

            @pl.when(i == 0)
            def _():
                plsc.subcore_barrier()
                pltpu.sync_copy(x_hbm.at[pl.ds(b * N + sid * PS, PS)],
                                spx.at[pl.ds(sid * PS, PS)])
                if REM:
                    @pl.when(sid == NS - 1)
                    def _():
                        pltpu.sync_copy(
                            x_hbm.at[pl.ds(b * N + NS * PS, REM)],
                            spx.at[pl.ds(NS * PS, REM)])
                plsc.subcore_barrier()
                for p in range(NBUF - 1):
                    gstart(p, xbs[p], gsems[p])

            for gi in range(2):
                g8 = 2 * i + gi

                @pl.when(2 * t + gi >= 2)
                def _():
                    pltpu.make_async_copy(
                        obs[gi], out_hbm.at[pl.ds(0, GOUT)],
                        osems[gi]).wait()

                for lp in range(GOUT // 2):
                    pair = i * GPB // 2 + gi * (GOUT // 2) + lp
                    par = (gi * (GOUT // 2) + lp) % NBUF

                    @pl.when(pair + NBUF - 1 < CHP)
                    def _():
                        gstart(pair + NBUF - 1, xbs[(par + NBUF - 1) % NBUF],
                               gsems[(par + NBUF - 1) % NBUF])

                    gwait(xbs[par], gsems[par])
                    for sub in range(2):
                        compute(pair * 2 + sub, lp * 2 + sub, sub * NH,
                                xbs[par], obs[gi])
                pltpu.make_async_copy(
                    obs[gi],
                    out_hbm.at[pl.ds(b * NPAD + n0 + g8 * GOUT, GOUT)],
                    osems[gi]).start()
            return carry

        lax.fori_loop(0, B * NI, tbody, 0)
        for gi in range(2):
            pltpu.make_async_copy(
                obs[gi], out_hbm.at[pl.ds(0, GOUT)], osems[gi]).wait()

    return sc_kernel


def kernel(x, coords_no, coords_decode, nh_indices):
    B, N, D = x.shape
    NH = nh_indices.shape[1]
    CH = -(-N // (NW * 2 * GOUT)) * 2 * GOUT
    NPAD = NW * CH

    nh = nh_indices.astype(jnp.int32)
    nh_p = jnp.zeros((NPAD, NH), jnp.int32).at[:N].set(nh).reshape(NPAD * NH)
    lon_no = coords_no[:, 0]
    lat_no = coords_no[:, 1]
    londe = jnp.zeros((NPAD,), jnp.float32).at[:N].set(coords_decode[:, 0])
    latde = jnp.zeros((NPAD,), jnp.float32).at[:N].set(coords_decode[:, 1])

    cosde, sinde = pl.pallas_call(
        _trig_tc_body,
        out_shape=(
            jax.ShapeDtypeStruct((NPAD // 128, 128), jnp.float32),
            jax.ShapeDtypeStruct((NPAD // 128, 128), jnp.float32),
        ),
    )(londe.reshape(NPAD // 128, 128))

    RB = 2048
    x32 = pl.pallas_call(
        _pack_tc_body,
        grid=(B * N // RB,),
        in_specs=[pl.BlockSpec((RB, D), lambda i: (i, 0))],
        out_specs=pl.BlockSpec((RB, D // 2), lambda i: (i, 0)),
        out_shape=jax.ShapeDtypeStruct((B * N, D // 2), jnp.int32),
    )(x.reshape(B * N, D))

    sc = _make_sc_kernel(B, N, D, NH, NPAD)
    out = sc(
        x32,
        nh_p,
        lon_no,
        lat_no,
        londe,
        latde,
        cosde.reshape(NPAD),
        sinde.reshape(NPAD),
    )
    return out.reshape(B, NPAD, D)[:, :N]
